# baseline (speedup 1.0000x reference)
import functools

import jax
import jax.numpy as jnp
from jax import lax
from jax.experimental import pallas as pl
from jax.experimental.pallas import tpu as pltpu
from jax.experimental.pallas import tpu_sc as plsc

_K = 8192
_D = 256
_N = 4096
_BT = 256
_BK = 2048
_NT = _N // _BT
_NK = _K // _BK
_CHUNK = 2048


def _tc_body(zf_ref, emb_ref, idx_ref, loss_ref, perp_ref,
             minv, mina, counts, loss_acc):
    i = pl.program_id(0)
    j = pl.program_id(1)

    @pl.when(jnp.logical_and(i == 0, j == 0))
    def _():
        counts[...] = jnp.zeros_like(counts)
        loss_acc[0, 0] = 0.0

    zf = zf_ref[...]
    emb = emb_ref[...]
    d = (jnp.sum(zf * zf, axis=1, keepdims=True)
         + jnp.sum(emb * emb, axis=1)[None, :]
         - 2.0 * lax.dot_general(zf, emb, (((1,), (1,)), ((), ())),
                                 preferred_element_type=jnp.float32))

    bmin = jnp.min(d, axis=1, keepdims=True)
    lane = lax.broadcasted_iota(jnp.int32, d.shape, 1)
    barg = jnp.min(jnp.where(d == bmin, lane, _K), axis=1,
                   keepdims=True) + j * _BK

    @pl.when(j == 0)
    def _():
        minv[...] = bmin
        mina[...] = barg

    @pl.when(j > 0)
    def _():
        better = bmin < minv[...]
        mina[...] = jnp.where(better, barg, mina[...])
        minv[...] = jnp.where(better, bmin, minv[...])

    @pl.when(j == _NK - 1)
    def _():
        arg = mina[...]
        idx_ref[...] = arg
        loss_acc[0, 0] += jnp.sum(minv[...])
        for c in range(0, _K, _CHUNK):
            lanec = lax.broadcasted_iota(jnp.int32, (_BT, _CHUNK), 1) + c
            onec = jnp.where(arg == lanec, 1.0, 0.0)
            counts[:, c:c + _CHUNK] += jnp.sum(onec, axis=0, keepdims=True)

        @pl.when(i == _NT - 1)
        def _():
            loss_ref[0, 0] = loss_acc[0, 0] / (_N * _D)
            avg = counts[...] / _N
            ent = jnp.sum(avg * jnp.log(avg + 1e-10))
            perp_ref[0, 0] = jnp.exp(-ent)


def _tc_distance_argmin(zf, embedding):
    return pl.pallas_call(
        _tc_body,
        grid=(_NT, _NK),
        in_specs=[
            pl.BlockSpec((_BT, _D), lambda i, j: (i, 0)),
            pl.BlockSpec((_BK, _D), lambda i, j: (j, 0)),
        ],
        out_specs=[
            pl.BlockSpec((_BT, 1), lambda i, j: (i, 0)),
            pl.BlockSpec(memory_space=pltpu.SMEM),
            pl.BlockSpec(memory_space=pltpu.SMEM),
        ],
        out_shape=[
            jax.ShapeDtypeStruct((_N, 1), jnp.int32),
            jax.ShapeDtypeStruct((1, 1), jnp.float32),
            jax.ShapeDtypeStruct((1, 1), jnp.float32),
        ],
        scratch_shapes=[
            pltpu.VMEM((_BT, 1), jnp.float32),
            pltpu.VMEM((_BT, 1), jnp.int32),
            pltpu.VMEM((1, _K), jnp.float32),
            pltpu.SMEM((1, 1), jnp.float32),
        ],
    )(zf, embedding)


def _sc_gather(embedding, idx):
    info = plsc.get_sparse_core_info()
    nc, ns = info.num_cores, info.num_subcores
    nw = nc * ns
    b_per_w = _N // nw
    mesh = plsc.VectorSubcoreMesh(core_axis_name="c", subcore_axis_name="s")

    @functools.partial(
        pl.kernel, mesh=mesh,
        out_type=jax.ShapeDtypeStruct((_N, _D), jnp.float32),
        scratch_types=[
            pltpu.VMEM((b_per_w,), jnp.int32),
            pltpu.VMEM((b_per_w, _D), jnp.float32),
            pltpu.SemaphoreType.DMA,
        ],
    )
    def k(table_hbm, idx_hbm, out_hbm, idx_v, rows_v, sem):
        wid = lax.axis_index("s") * nc + lax.axis_index("c")
        base = wid * b_per_w
        pltpu.sync_copy(idx_hbm.at[pl.ds(base, b_per_w)], idx_v)
        pltpu.async_copy(table_hbm.at[idx_v], rows_v, sem).wait()
        pltpu.sync_copy(rows_v, out_hbm.at[pl.ds(base, b_per_w)])

    return k(embedding, idx)


def kernel(z, embedding):
    B, D, T = z.shape
    zf = jnp.transpose(z, (0, 2, 1)).reshape(-1, D)
    idx2, loss, perp = _tc_distance_argmin(zf, embedding)
    z_vq = _sc_gather(embedding, idx2.reshape(-1))
    z_out = jnp.transpose(z_vq.reshape(B, T, D), (0, 2, 1))
    scalar_loss = loss[0, 0]
    return (z_out, scalar_loss, scalar_loss, perp[0, 0])


# speedup vs baseline: 1.1928x; 1.1928x over previous
import functools

import jax
import jax.numpy as jnp
from jax import lax
from jax.experimental import pallas as pl
from jax.experimental.pallas import tpu as pltpu
from jax.experimental.pallas import tpu_sc as plsc

_K = 8192
_D = 256
_T = 256
_N = 4096
_BK = 2048
_NT = _N // _T
_NK = _K // _BK
_CK = 512


def _tc_body(z_ref, emb_ref, idx_ref, loss_ref, perp_ref,
             minv_all, mina_all, counts, loss_acc):
    j = pl.program_id(0)
    i = pl.program_id(1)

    @pl.when(jnp.logical_and(j == 0, i == 0))
    def _():
        counts[...] = jnp.zeros_like(counts)
        loss_acc[0, 0] = 0.0

    zb = z_ref[0]
    emb = emb_ref[...]
    d = (jnp.sum(zb * zb, axis=0, keepdims=True)
         + jnp.sum(emb * emb, axis=1, keepdims=True)
         - 2.0 * lax.dot_general(emb, zb, (((1,), (0,)), ((), ())),
                                 preferred_element_type=jnp.float32))

    bmin = jnp.min(d, axis=0, keepdims=True)
    srow = lax.broadcasted_iota(jnp.int32, d.shape, 0)
    barg = jnp.min(jnp.where(d == bmin, srow, _K), axis=0,
                   keepdims=True) + j * _BK

    @pl.when(j == 0)
    def _():
        minv_all[i] = bmin
        mina_all[i] = barg

    @pl.when(j > 0)
    def _():
        prev = minv_all[i]
        better = bmin < prev
        mina_all[i] = jnp.where(better, barg, mina_all[i])
        minv_all[i] = jnp.where(better, bmin, prev)

    @pl.when(j == _NK - 1)
    def _():
        arg = mina_all[i]
        idx_ref[0] = arg
        loss_acc[0, 0] += jnp.sum(minv_all[i])
        for c in range(0, _K, _CK):
            krow = lax.broadcasted_iota(jnp.int32, (_CK, _T), 0) + c
            onec = jnp.where(arg == krow, 1.0, 0.0)
            counts[c:c + _CK, :] += jnp.sum(onec, axis=1, keepdims=True)

        @pl.when(i == _NT - 1)
        def _():
            loss_ref[0, 0] = loss_acc[0, 0] / (_N * _D)
            avg = counts[...] / _N
            ent = jnp.sum(avg * jnp.log(avg + 1e-10))
            perp_ref[0, 0] = jnp.exp(-ent)


def _tc_distance_argmin(z, embedding):
    return pl.pallas_call(
        _tc_body,
        grid=(_NK, _NT),
        in_specs=[
            pl.BlockSpec((1, _D, _T), lambda j, i: (i, 0, 0)),
            pl.BlockSpec((_BK, _D), lambda j, i: (j, 0)),
        ],
        out_specs=[
            pl.BlockSpec((1, 1, _T), lambda j, i: (i, 0, 0)),
            pl.BlockSpec(memory_space=pltpu.SMEM),
            pl.BlockSpec(memory_space=pltpu.SMEM),
        ],
        out_shape=[
            jax.ShapeDtypeStruct((_NT, 1, _T), jnp.int32),
            jax.ShapeDtypeStruct((1, 1), jnp.float32),
            jax.ShapeDtypeStruct((1, 1), jnp.float32),
        ],
        scratch_shapes=[
            pltpu.VMEM((_NT, 1, _T), jnp.float32),
            pltpu.VMEM((_NT, 1, _T), jnp.int32),
            pltpu.VMEM((_K, 1), jnp.float32),
            pltpu.SMEM((1, 1), jnp.float32),
        ],
    )(z, embedding)


def _sc_gather(embedding, idx):
    info = plsc.get_sparse_core_info()
    nc, ns = info.num_cores, info.num_subcores
    nw = nc * ns
    b_per_w = _N // nw
    mesh = plsc.VectorSubcoreMesh(core_axis_name="c", subcore_axis_name="s")

    @functools.partial(
        pl.kernel, mesh=mesh,
        out_type=jax.ShapeDtypeStruct((_N, _D), jnp.float32),
        scratch_types=[
            pltpu.VMEM((b_per_w,), jnp.int32),
            pltpu.VMEM((b_per_w, _D), jnp.float32),
            pltpu.SemaphoreType.DMA,
        ],
    )
    def k(table_hbm, idx_hbm, out_hbm, idx_v, rows_v, sem):
        wid = lax.axis_index("s") * nc + lax.axis_index("c")
        base = wid * b_per_w
        pltpu.sync_copy(idx_hbm.at[pl.ds(base, b_per_w)], idx_v)
        pltpu.async_copy(table_hbm.at[idx_v], rows_v, sem).wait()
        pltpu.sync_copy(rows_v, out_hbm.at[pl.ds(base, b_per_w)])

    return k(embedding, idx)


def kernel(z, embedding):
    B, D, T = z.shape
    idx2, loss, perp = _tc_distance_argmin(z, embedding)
    z_vq = _sc_gather(embedding, idx2.reshape(-1))
    z_out = jnp.transpose(z_vq.reshape(B, T, D), (0, 2, 1))
    scalar_loss = loss[0, 0]
    return (z_out, scalar_loss, scalar_loss, perp[0, 0])


# speedup vs baseline: 1.4081x; 1.1805x over previous
import functools

import jax
import jax.numpy as jnp
from jax import lax
from jax.experimental import pallas as pl
from jax.experimental.pallas import tpu as pltpu
from jax.experimental.pallas import tpu_sc as plsc

_K = 8192
_D = 256
_T = 256
_N = 4096
_BK = 2048
_NT = _N // _T
_NK = _K // _BK


def _tc_body(z_ref, emb_ref, idx_ref, loss_ref,
             minv_all, mina_all, emb_m2, esq_s, loss_acc):
    j = pl.program_id(0)
    i = pl.program_id(1)

    @pl.when(jnp.logical_and(j == 0, i == 0))
    def _():
        loss_acc[0, 0] = 0.0

    @pl.when(i == 0)
    def _():
        emb = emb_ref[...]
        emb_m2[...] = emb * -2.0
        esq_s[...] = jnp.sum(emb * emb, axis=1, keepdims=True)

    zb = z_ref[0]
    zsq = jnp.sum(zb * zb, axis=0, keepdims=True)
    mm2 = lax.dot_general(emb_m2[...], zb, (((1,), (0,)), ((), ())),
                          preferred_element_type=jnp.float32)
    d = (zsq + esq_s[...]) + mm2

    bmin = jnp.min(d, axis=0, keepdims=True)
    srow = lax.broadcasted_iota(jnp.int32, d.shape, 0)
    barg = jnp.min(jnp.where(d == bmin, srow, _K), axis=0,
                   keepdims=True) + j * _BK

    @pl.when(j == 0)
    def _():
        minv_all[i] = bmin
        mina_all[i] = barg

    @pl.when(j > 0)
    def _():
        prev = minv_all[i]
        better = bmin < prev
        mina_all[i] = jnp.where(better, barg, mina_all[i])
        minv_all[i] = jnp.where(better, bmin, prev)

    @pl.when(j == _NK - 1)
    def _():
        idx_ref[0] = mina_all[i]
        loss_acc[0, 0] += jnp.sum(minv_all[i])

        @pl.when(i == _NT - 1)
        def _():
            loss_ref[0, 0] = loss_acc[0, 0] / (_N * _D)


def _tc_distance_argmin(z, embedding):
    return pl.pallas_call(
        _tc_body,
        grid=(_NK, _NT),
        in_specs=[
            pl.BlockSpec((1, _D, _T), lambda j, i: (i, 0, 0)),
            pl.BlockSpec((_BK, _D), lambda j, i: (j, 0)),
        ],
        out_specs=[
            pl.BlockSpec((1, 1, _T), lambda j, i: (i, 0, 0)),
            pl.BlockSpec(memory_space=pltpu.SMEM),
        ],
        out_shape=[
            jax.ShapeDtypeStruct((_NT, 1, _T), jnp.int32),
            jax.ShapeDtypeStruct((1, 1), jnp.float32),
        ],
        scratch_shapes=[
            pltpu.VMEM((_NT, 1, _T), jnp.float32),
            pltpu.VMEM((_NT, 1, _T), jnp.int32),
            pltpu.VMEM((_BK, _D), jnp.float32),
            pltpu.VMEM((_BK, 1), jnp.float32),
            pltpu.SMEM((1, 1), jnp.float32),
        ],
    )(z, embedding)


def _sc_gather_hist(embedding, idx):
    info = plsc.get_sparse_core_info()
    nc, ns, nl = info.num_cores, info.num_subcores, info.num_lanes
    b_per_w = _N // (nc * ns)
    mesh = plsc.VectorSubcoreMesh(core_axis_name="c", subcore_axis_name="s")

    @functools.partial(
        pl.kernel, mesh=mesh,
        out_type=[
            jax.ShapeDtypeStruct((_N, _D), jnp.float32),
            jax.ShapeDtypeStruct((nc, _K), jnp.float32),
        ],
        scratch_types=[
            pltpu.VMEM((b_per_w,), jnp.int32),
            pltpu.VMEM((b_per_w, _D), jnp.float32),
            pltpu.VMEM((b_per_w,), jnp.float32),
            pltpu.VMEM((_K,), jnp.float32),
            pltpu.VMEM_SHARED((_K,), jnp.float32),
            pltpu.SemaphoreType.DMA,
        ],
    )
    def k(table_hbm, idx_hbm, out_hbm, hist_hbm,
          idx_v, rows_v, ones_v, zer_v, hist_s, sem):
        cidx = lax.axis_index("c")
        sidx = lax.axis_index("s")
        wid = sidx * nc + cidx
        base = wid * b_per_w
        pltpu.sync_copy(idx_hbm.at[pl.ds(base, b_per_w)], idx_v)
        cp = pltpu.async_copy(table_hbm.at[idx_v], rows_v, sem)

        ones = jnp.full((nl,), 1.0, jnp.float32)

        def obody(g, carry):
            ones_v[pl.ds(g * nl, nl)] = ones
            return carry

        lax.fori_loop(0, b_per_w // nl, obody, 0)

        @pl.when(sidx == 0)
        def _():
            zeros = jnp.zeros((nl,), jnp.float32)

            def zbody(g, carry):
                zer_v[pl.ds(g * nl, nl)] = zeros
                return carry

            lax.fori_loop(0, _K // nl, zbody, 0)
            pltpu.sync_copy(zer_v, hist_s)

        plsc.subcore_barrier()
        pltpu.sync_copy(ones_v, hist_s.at[idx_v], add=True)
        plsc.subcore_barrier()

        @pl.when(sidx == 0)
        def _():
            pltpu.sync_copy(hist_s, hist_hbm.at[cidx])

        cp.wait()
        pltpu.sync_copy(rows_v, out_hbm.at[pl.ds(base, b_per_w)])

    return k(embedding, idx)


def _fin_body(h_ref, perp_ref):
    counts = jnp.sum(h_ref[...], axis=0, keepdims=True)
    avg = counts / _N
    ent = jnp.sum(avg * jnp.log(avg + 1e-10))
    perp_ref[0, 0] = jnp.exp(-ent)


def _finalize_perp(hists):
    return pl.pallas_call(
        _fin_body,
        out_specs=pl.BlockSpec(memory_space=pltpu.SMEM),
        out_shape=jax.ShapeDtypeStruct((1, 1), jnp.float32),
    )(hists)


def kernel(z, embedding):
    B, D, T = z.shape
    idx3, loss = _tc_distance_argmin(z, embedding)
    z_vq, hists = _sc_gather_hist(embedding, idx3.reshape(-1))
    perp = _finalize_perp(hists)
    z_out = jnp.transpose(z_vq.reshape(B, T, D), (0, 2, 1))
    scalar_loss = loss[0, 0]
    return (z_out, scalar_loss, scalar_loss, perp[0, 0])


# speedup vs baseline: 1.4824x; 1.0527x over previous
import functools

import jax
import jax.numpy as jnp
from jax import lax
from jax.experimental import pallas as pl
from jax.experimental.pallas import tpu as pltpu
from jax.experimental.pallas import tpu_sc as plsc

_K = 8192
_D = 256
_T = 256
_N = 4096
_BK = 2048
_NT = _N // _T
_NK = _K // _BK


def _tc_body(z_ref, emb_ref, idx_ref, loss_ref,
             minv_all, mina_all, emb_m2, esq_s, loss_acc):
    j = pl.program_id(0)
    i = pl.program_id(1)

    @pl.when(jnp.logical_and(j == 0, i == 0))
    def _():
        loss_acc[0, 0] = 0.0

    @pl.when(i == 0)
    def _():
        emb = emb_ref[...]
        emb_m2[...] = emb * -2.0
        esq_s[...] = jnp.sum(emb * emb, axis=1, keepdims=True)

    zb = z_ref[0]
    zsq = jnp.sum(zb * zb, axis=0, keepdims=True)
    mm2 = lax.dot_general(emb_m2[...], zb, (((1,), (0,)), ((), ())),
                          preferred_element_type=jnp.float32)
    d = (zsq + esq_s[...]) + mm2

    bmin = jnp.min(d, axis=0, keepdims=True)
    barg = jnp.argmin(d, axis=0).astype(jnp.int32).reshape(1, _T) + j * _BK

    @pl.when(j == 0)
    def _():
        minv_all[i] = bmin
        mina_all[i] = barg

    @pl.when(j > 0)
    def _():
        prev = minv_all[i]
        better = bmin < prev
        mina_all[i] = jnp.where(better, barg, mina_all[i])
        minv_all[i] = jnp.where(better, bmin, prev)

    @pl.when(j == _NK - 1)
    def _():
        idx_ref[0] = mina_all[i]
        loss_acc[0, 0] += jnp.sum(minv_all[i])

        @pl.when(i == _NT - 1)
        def _():
            loss_ref[0, 0] = loss_acc[0, 0] / (_N * _D)


def _tc_distance_argmin(z, embedding):
    return pl.pallas_call(
        _tc_body,
        grid=(_NK, _NT),
        in_specs=[
            pl.BlockSpec((1, _D, _T), lambda j, i: (i, 0, 0)),
            pl.BlockSpec((_BK, _D), lambda j, i: (j, 0)),
        ],
        out_specs=[
            pl.BlockSpec((1, 1, _T), lambda j, i: (i, 0, 0)),
            pl.BlockSpec(memory_space=pltpu.SMEM),
        ],
        out_shape=[
            jax.ShapeDtypeStruct((_NT, 1, _T), jnp.int32),
            jax.ShapeDtypeStruct((1, 1), jnp.float32),
        ],
        scratch_shapes=[
            pltpu.VMEM((_NT, 1, _T), jnp.float32),
            pltpu.VMEM((_NT, 1, _T), jnp.int32),
            pltpu.VMEM((_BK, _D), jnp.float32),
            pltpu.VMEM((_BK, 1), jnp.float32),
            pltpu.SMEM((1, 1), jnp.float32),
        ],
    )(z, embedding)


def _sc_gather_hist(embedding, idx):
    info = plsc.get_sparse_core_info()
    nc, ns, nl = info.num_cores, info.num_subcores, info.num_lanes
    b_per_w = _N // (nc * ns)
    mesh = plsc.VectorSubcoreMesh(core_axis_name="c", subcore_axis_name="s")

    @functools.partial(
        pl.kernel, mesh=mesh,
        out_type=[
            jax.ShapeDtypeStruct((_N, _D), jnp.float32),
            jax.ShapeDtypeStruct((nc, _K), jnp.float32),
        ],
        scratch_types=[
            pltpu.VMEM((b_per_w,), jnp.int32),
            pltpu.VMEM((b_per_w, _D), jnp.float32),
            pltpu.VMEM((b_per_w,), jnp.float32),
            pltpu.VMEM((_K,), jnp.float32),
            pltpu.VMEM_SHARED((_K,), jnp.float32),
            pltpu.SemaphoreType.DMA,
        ],
    )
    def k(table_hbm, idx_hbm, out_hbm, hist_hbm,
          idx_v, rows_v, ones_v, zer_v, hist_s, sem):
        cidx = lax.axis_index("c")
        sidx = lax.axis_index("s")
        wid = sidx * nc + cidx
        base = wid * b_per_w
        pltpu.sync_copy(idx_hbm.at[pl.ds(base, b_per_w)], idx_v)
        cp = pltpu.async_copy(table_hbm.at[idx_v], rows_v, sem)

        ones = jnp.full((nl,), 1.0, jnp.float32)

        def obody(g, carry):
            ones_v[pl.ds(g * nl, nl)] = ones
            return carry

        lax.fori_loop(0, b_per_w // nl, obody, 0)

        @pl.when(sidx == 0)
        def _():
            zeros = jnp.zeros((nl,), jnp.float32)

            def zbody(g, carry):
                zer_v[pl.ds(g * nl, nl)] = zeros
                return carry

            lax.fori_loop(0, _K // nl, zbody, 0)
            pltpu.sync_copy(zer_v, hist_s)

        plsc.subcore_barrier()
        pltpu.sync_copy(ones_v, hist_s.at[idx_v], add=True)
        plsc.subcore_barrier()

        @pl.when(sidx == 0)
        def _():
            pltpu.sync_copy(hist_s, hist_hbm.at[cidx])

        cp.wait()
        pltpu.sync_copy(rows_v, out_hbm.at[pl.ds(base, b_per_w)])

    return k(embedding, idx)


def _fin_body(h_ref, perp_ref):
    counts = jnp.sum(h_ref[...], axis=0, keepdims=True)
    avg = counts / _N
    ent = jnp.sum(avg * jnp.log(avg + 1e-10))
    perp_ref[0, 0] = jnp.exp(-ent)


def _finalize_perp(hists):
    return pl.pallas_call(
        _fin_body,
        out_specs=pl.BlockSpec(memory_space=pltpu.SMEM),
        out_shape=jax.ShapeDtypeStruct((1, 1), jnp.float32),
    )(hists)


def kernel(z, embedding):
    B, D, T = z.shape
    idx3, loss = _tc_distance_argmin(z, embedding)
    z_vq, hists = _sc_gather_hist(embedding, idx3.reshape(-1))
    perp = _finalize_perp(hists)
    z_out = jnp.transpose(z_vq.reshape(B, T, D), (0, 2, 1))
    scalar_loss = loss[0, 0]
    return (z_out, scalar_loss, scalar_loss, perp[0, 0])
